# Initial kernel scaffold; baseline (speedup 1.0000x reference)
import jax
import jax.numpy as jnp
from jax.experimental import pallas as pl


def kernel(x, edge_index, batch, W1, a_src1, a_dst1, b1, W2, a_src2, a_dst2, b2, Wc1, bc1, Wc2, bc2):
    raise NotImplementedError("write your pallas kernel here")


# speedup vs baseline: 21.9175x; 21.9175x over previous
import functools

import jax
import jax.numpy as jnp
from jax import lax
from jax.experimental import pallas as pl
from jax.experimental.pallas import tpu as pltpu
from jax.experimental.pallas import tpu_sc as plsc

NC = 2
NS = 16
L = 16
NW = NC * NS
G = 64
BLK = 2000


def _leaky(z):
    return jnp.maximum(z, 0.2 * z)


def _elu(o):
    return jnp.where(o > 0, o, jnp.exp(o) - 1.0)


def _dot(a, b):
    return jnp.dot(a, b, preferred_element_type=jnp.float32)


def _proj1_body(x_ref, w_ref, asrc_ref, adst_ref, h_ref, as_ref, ad_ref):
    h = _dot(x_ref[...], w_ref[...])
    h_ref[...] = h
    as_ref[...] = _dot(h, asrc_ref[...])
    ad_ref[...] = _dot(h, adst_ref[...])


def _proj1(x, W, asrc, adst):
    n, d = x.shape
    hw = W.shape[1]
    return pl.pallas_call(
        _proj1_body,
        grid=(n // BLK,),
        in_specs=[
            pl.BlockSpec((BLK, d), lambda i: (i, 0)),
            pl.BlockSpec((d, hw), lambda i: (0, 0)),
            pl.BlockSpec((hw, 1), lambda i: (0, 0)),
            pl.BlockSpec((hw, 1), lambda i: (0, 0)),
        ],
        out_specs=[
            pl.BlockSpec((BLK, hw), lambda i: (i, 0)),
            pl.BlockSpec((BLK, 1), lambda i: (i, 0)),
            pl.BlockSpec((BLK, 1), lambda i: (i, 0)),
        ],
        out_shape=[
            jax.ShapeDtypeStruct((n, hw), jnp.float32),
            jax.ShapeDtypeStruct((n, 1), jnp.float32),
            jax.ShapeDtypeStruct((n, 1), jnp.float32),
        ],
    )(x, W, asrc, adst)


def _proj2_body(acc_ref, dent_ref, hprev_ref, exself_ref, b_ref, w_ref,
                asrc_ref, adst_ref, h_ref, as_ref, ad_ref):
    a = acc_ref[0] + acc_ref[1] + exself_ref[...] * hprev_ref[...]
    den = jnp.sum(dent_ref[...], axis=1, keepdims=True) + exself_ref[...]
    o = a / (den + 1e-16) + b_ref[...]
    g = _elu(o)
    h = _dot(g, w_ref[...])
    h_ref[...] = h
    as_ref[...] = _dot(h, asrc_ref[...])
    ad_ref[...] = _dot(h, adst_ref[...])


def _proj2(acc, dent, hprev, exself, b, W, asrc, adst):
    n, hw = hprev.shape
    return pl.pallas_call(
        _proj2_body,
        grid=(n // BLK,),
        in_specs=[
            pl.BlockSpec((2, BLK, hw), lambda i: (0, i, 0)),
            pl.BlockSpec((BLK, 2), lambda i: (i, 0)),
            pl.BlockSpec((BLK, hw), lambda i: (i, 0)),
            pl.BlockSpec((BLK, 1), lambda i: (i, 0)),
            pl.BlockSpec((1, hw), lambda i: (0, 0)),
            pl.BlockSpec((hw, hw), lambda i: (0, 0)),
            pl.BlockSpec((hw, 1), lambda i: (0, 0)),
            pl.BlockSpec((hw, 1), lambda i: (0, 0)),
        ],
        out_specs=[
            pl.BlockSpec((BLK, hw), lambda i: (i, 0)),
            pl.BlockSpec((BLK, 1), lambda i: (i, 0)),
            pl.BlockSpec((BLK, 1), lambda i: (i, 0)),
        ],
        out_shape=[
            jax.ShapeDtypeStruct((n, hw), jnp.float32),
            jax.ShapeDtypeStruct((n, 1), jnp.float32),
            jax.ShapeDtypeStruct((n, 1), jnp.float32),
        ],
    )(acc, dent, hprev, exself, b, W, asrc, adst)


def _selfw_body(as_ref, ad_ref, m_ref, exself_ref):
    a_s = as_ref[...]
    a_d = ad_ref[...]
    m = _leaky(jnp.max(a_s) + jnp.max(a_d))
    m_ref[...] = jnp.full((1, 1), 0.0, jnp.float32) + m
    exself_ref[...] = jnp.exp(_leaky(a_s + a_d) - m)


def _selfw(a_s, a_d):
    n = a_s.shape[0]
    return pl.pallas_call(
        _selfw_body,
        out_shape=[
            jax.ShapeDtypeStruct((1, 1), jnp.float32),
            jax.ShapeDtypeStruct((n, 1), jnp.float32),
        ],
    )(a_s, a_d)


def _pool_body(acc_ref, dent_ref, hprev_ref, exself_ref, b_ref, batch_ref,
               pooled_ref, cnts_ref):
    a = acc_ref[0] + acc_ref[1] + exself_ref[...] * hprev_ref[...]
    den = jnp.sum(dent_ref[...], axis=1, keepdims=True) + exself_ref[...]
    o = a / (den + 1e-16) + b_ref[...]
    g = _elu(o)
    gi = lax.broadcasted_iota(jnp.int32, (BLK, G), 1)
    maskT = jnp.where(gi == batch_ref[...], 1.0, 0.0)

    @pl.when(pl.program_id(0) == 0)
    def _():
        pooled_ref[...] = jnp.zeros_like(pooled_ref)
        cnts_ref[...] = jnp.zeros_like(cnts_ref)

    dnum = (((0,), (0,)), ((), ()))
    pooled_ref[...] += lax.dot_general(
        maskT, g, dnum, precision=lax.Precision.HIGHEST,
        preferred_element_type=jnp.float32)
    cnts_ref[...] += lax.dot_general(
        maskT, jnp.ones((BLK, 1), jnp.float32), dnum,
        precision=lax.Precision.HIGHEST,
        preferred_element_type=jnp.float32)


def _pool(acc, dent, hprev, exself, b, batch2d):
    n, hw = hprev.shape
    return pl.pallas_call(
        _pool_body,
        grid=(n // BLK,),
        in_specs=[
            pl.BlockSpec((2, BLK, hw), lambda i: (0, i, 0)),
            pl.BlockSpec((BLK, 2), lambda i: (i, 0)),
            pl.BlockSpec((BLK, hw), lambda i: (i, 0)),
            pl.BlockSpec((BLK, 1), lambda i: (i, 0)),
            pl.BlockSpec((1, hw), lambda i: (0, 0)),
            pl.BlockSpec((BLK, 1), lambda i: (i, 0)),
        ],
        out_specs=[
            pl.BlockSpec((G, hw), lambda i: (0, 0)),
            pl.BlockSpec((G, 1), lambda i: (0, 0)),
        ],
        out_shape=[
            jax.ShapeDtypeStruct((G, hw), jnp.float32),
            jax.ShapeDtypeStruct((G, 1), jnp.float32),
        ],
    )(acc, dent, hprev, exself, b, batch2d)


def _head_body(pooled_ref, cnts_ref, wc1_ref, bc1_ref, wc2_ref, bc2_ref,
               out_ref):
    mean = pooled_ref[...] / jnp.maximum(cnts_ref[...], 1.0)
    z = jnp.maximum(_dot(mean, wc1_ref[...]) + bc1_ref[...], 0.0)
    out_ref[...] = _dot(z, wc2_ref[...]) + bc2_ref[...]


def _head(pooled, cnts, Wc1, bc1, Wc2, bc2):
    return pl.pallas_call(
        _head_body,
        out_shape=jax.ShapeDtypeStruct((G, Wc2.shape[1]), jnp.float32),
    )(pooled, cnts, Wc1, bc1, Wc2, bc2)


def _sc_edge(h, src, dst, a_s, a_d, m16, z2, z1):
    n, hw = h.shape
    e = src.shape[0]
    epw = e // NW
    ch = 80
    nchunk = epw // ch

    def body(h_hbm, src_hbm, dst_hbm, as_hbm, ad_hbm, m_hbm, z2_hbm, z1_hbm,
             acc_out, den_out,
             asv, adv, mvv, srcv, dstv, exv, rows, acc_sh, den_sh):
        ci = lax.axis_index("c")
        si = lax.axis_index("s")

        @pl.when(si == 0)
        def _():
            pltpu.sync_copy(z2_hbm, acc_sh)
            pltpu.sync_copy(z1_hbm, den_sh)

        pltpu.sync_copy(as_hbm, asv)
        pltpu.sync_copy(ad_hbm, adv)
        pltpu.sync_copy(m_hbm, mvv)
        plsc.subcore_barrier()
        mvec = mvv[...]
        wid = ci * NS + si

        @pl.loop(0, nchunk)
        def _chunk(c):
            base = wid * epw + c * ch
            pltpu.sync_copy(src_hbm.at[pl.ds(base, ch)], srcv)
            pltpu.sync_copy(dst_hbm.at[pl.ds(base, ch)], dstv)
            pltpu.sync_copy(h_hbm.at[srcv], rows)

            @pl.loop(0, ch // L)
            def _ex(i):
                s16 = srcv[pl.ds(i * L, L)]
                d16 = dstv[pl.ds(i * L, L)]
                z = plsc.load_gather(asv, [s16]) + plsc.load_gather(adv, [d16])
                exv[pl.ds(i * L, L)] = jnp.exp(jnp.maximum(z, 0.2 * z) - mvec)

            @pl.loop(0, ch // L)
            def _mul(i):
                ex16 = exv[pl.ds(i * L, L)]
                for k in range(L):
                    exb = jnp.full((L,), 0.0, jnp.float32) + ex16[k]
                    idx = i * L + k
                    for j in range(hw // L):
                        rows[idx, pl.ds(j * L, L)] = (
                            rows[idx, pl.ds(j * L, L)] * exb)

            pltpu.sync_copy(rows, acc_sh.at[dstv], add=True)
            pltpu.sync_copy(exv, den_sh.at[dstv], add=True)

        plsc.subcore_barrier()

        @pl.when(si == 0)
        def _():
            pltpu.sync_copy(acc_sh, acc_out.at[ci])
            pltpu.sync_copy(den_sh, den_out.at[ci])

    f = pl.kernel(
        body,
        out_type=(
            jax.ShapeDtypeStruct((NC, n, hw), jnp.float32),
            jax.ShapeDtypeStruct((NC, n), jnp.float32),
        ),
        mesh=plsc.VectorSubcoreMesh(core_axis_name="c", subcore_axis_name="s"),
        compiler_params=pltpu.CompilerParams(
            needs_layout_passes=False, use_tc_tiling_on_sc=False),
        scratch_types=[
            pltpu.VMEM((n,), jnp.float32),
            pltpu.VMEM((n,), jnp.float32),
            pltpu.VMEM((L,), jnp.float32),
            pltpu.VMEM((ch,), jnp.int32),
            pltpu.VMEM((ch,), jnp.int32),
            pltpu.VMEM((ch,), jnp.float32),
            pltpu.VMEM((ch, hw), jnp.float32),
            pltpu.VMEM_SHARED((n, hw), jnp.float32),
            pltpu.VMEM_SHARED((n,), jnp.float32),
        ],
    )
    return f(h, src, dst, a_s, a_d, m16, z2, z1)


def kernel(x, edge_index, batch, W1, a_src1, a_dst1, b1,
           W2, a_src2, a_dst2, b2, Wc1, bc1, Wc2, bc2):
    n, d = x.shape
    hw = W1.shape[1]
    src = edge_index[0]
    dst = edge_index[1]
    z2 = jnp.zeros((n, hw), jnp.float32)
    z1 = jnp.zeros((n,), jnp.float32)

    h1, as1, ad1 = _proj1(x, W1, a_src1.reshape(hw, 1), a_dst1.reshape(hw, 1))
    m1, exself1 = _selfw(as1, ad1)
    m16 = jnp.broadcast_to(m1.reshape(1), (L,))
    acc1, den1 = _sc_edge(h1, src, dst, as1.reshape(n), ad1.reshape(n),
                          m16, z2, z1)
    h2, as2, ad2 = _proj2(acc1, den1.T, h1, exself1, b1.reshape(1, hw),
                          W2, a_src2.reshape(hw, 1), a_dst2.reshape(hw, 1))
    m2, exself2 = _selfw(as2, ad2)
    m16b = jnp.broadcast_to(m2.reshape(1), (L,))
    acc2, den2 = _sc_edge(h2, src, dst, as2.reshape(n), ad2.reshape(n),
                          m16b, z2, z1)
    pooled, cnts = _pool(acc2, den2.T, h2, exself2, b2.reshape(1, hw),
                         batch.reshape(n, 1))
    return _head(pooled, cnts, Wc1, bc1.reshape(1, -1), Wc2, bc2.reshape(1, 1))


# speedup vs baseline: 40.7211x; 1.8579x over previous
import functools

import jax
import jax.numpy as jnp
from jax import lax
from jax.experimental import pallas as pl
from jax.experimental.pallas import tpu as pltpu
from jax.experimental.pallas import tpu_sc as plsc

NC = 2
NS = 16
L = 16
NW = NC * NS
G = 64
BLK = 2000


def _leaky(z):
    return jnp.maximum(z, 0.2 * z)


def _elu(o):
    return jnp.where(o > 0, o, jnp.exp(o) - 1.0)


def _dot(a, b):
    return jnp.dot(a, b, preferred_element_type=jnp.float32)


def _proj1_body(x_ref, w_ref, asrc_ref, adst_ref, h_ref, as_ref, ad_ref):
    h = _dot(x_ref[...], w_ref[...])
    h_ref[...] = h
    as_ref[...] = _dot(h, asrc_ref[...])
    ad_ref[...] = _dot(h, adst_ref[...])


def _proj1(x, W, asrc, adst):
    n, d = x.shape
    hw = W.shape[1]
    return pl.pallas_call(
        _proj1_body,
        grid=(n // BLK,),
        in_specs=[
            pl.BlockSpec((BLK, d), lambda i: (i, 0)),
            pl.BlockSpec((d, hw), lambda i: (0, 0)),
            pl.BlockSpec((hw, 1), lambda i: (0, 0)),
            pl.BlockSpec((hw, 1), lambda i: (0, 0)),
        ],
        out_specs=[
            pl.BlockSpec((BLK, hw), lambda i: (i, 0)),
            pl.BlockSpec((BLK, 1), lambda i: (i, 0)),
            pl.BlockSpec((BLK, 1), lambda i: (i, 0)),
        ],
        out_shape=[
            jax.ShapeDtypeStruct((n, hw), jnp.float32),
            jax.ShapeDtypeStruct((n, 1), jnp.float32),
            jax.ShapeDtypeStruct((n, 1), jnp.float32),
        ],
    )(x, W, asrc, adst)


def _proj2_body(acc_ref, hprev_ref, exself_ref, b_ref, w_ref,
                asrc_ref, adst_ref, h_ref, as_ref, ad_ref):
    hw = hprev_ref.shape[1]
    a = (acc_ref[0, :, 0:hw] + acc_ref[1, :, 0:hw]
         + exself_ref[...] * hprev_ref[...])
    den = (acc_ref[0, :, hw:hw + 1] + acc_ref[1, :, hw:hw + 1]
           + exself_ref[...])
    o = a / (den + 1e-16) + b_ref[...]
    g = _elu(o)
    h = _dot(g, w_ref[...])
    h_ref[...] = h
    as_ref[...] = _dot(h, asrc_ref[...])
    ad_ref[...] = _dot(h, adst_ref[...])


def _proj2(acc, hprev, exself, b, W, asrc, adst):
    n, hw = hprev.shape
    return pl.pallas_call(
        _proj2_body,
        grid=(n // BLK,),
        in_specs=[
            pl.BlockSpec((2, BLK, hw + L), lambda i: (0, i, 0)),
            pl.BlockSpec((BLK, hw), lambda i: (i, 0)),
            pl.BlockSpec((BLK, 1), lambda i: (i, 0)),
            pl.BlockSpec((1, hw), lambda i: (0, 0)),
            pl.BlockSpec((hw, hw), lambda i: (0, 0)),
            pl.BlockSpec((hw, 1), lambda i: (0, 0)),
            pl.BlockSpec((hw, 1), lambda i: (0, 0)),
        ],
        out_specs=[
            pl.BlockSpec((BLK, hw), lambda i: (i, 0)),
            pl.BlockSpec((BLK, 1), lambda i: (i, 0)),
            pl.BlockSpec((BLK, 1), lambda i: (i, 0)),
        ],
        out_shape=[
            jax.ShapeDtypeStruct((n, hw), jnp.float32),
            jax.ShapeDtypeStruct((n, 1), jnp.float32),
            jax.ShapeDtypeStruct((n, 1), jnp.float32),
        ],
    )(acc, hprev, exself, b, W, asrc, adst)


def _selfw_body(as_ref, ad_ref, m_ref, exself_ref):
    a_s = as_ref[...]
    a_d = ad_ref[...]
    m = _leaky(jnp.max(a_s) + jnp.max(a_d))
    m_ref[...] = jnp.full((1, 1), 0.0, jnp.float32) + m
    exself_ref[...] = jnp.exp(_leaky(a_s + a_d) - m)


def _selfw(a_s, a_d):
    n = a_s.shape[0]
    return pl.pallas_call(
        _selfw_body,
        out_shape=[
            jax.ShapeDtypeStruct((1, 1), jnp.float32),
            jax.ShapeDtypeStruct((n, 1), jnp.float32),
        ],
    )(a_s, a_d)


def _pool_body(acc_ref, hprev_ref, exself_ref, b_ref, batch_ref,
               pooled_ref, cnts_ref):
    hw = hprev_ref.shape[1]
    a = (acc_ref[0, :, 0:hw] + acc_ref[1, :, 0:hw]
         + exself_ref[...] * hprev_ref[...])
    den = (acc_ref[0, :, hw:hw + 1] + acc_ref[1, :, hw:hw + 1]
           + exself_ref[...])
    o = a / (den + 1e-16) + b_ref[...]
    g = _elu(o)
    gi = lax.broadcasted_iota(jnp.int32, (BLK, G), 1)
    maskT = jnp.where(gi == batch_ref[...], 1.0, 0.0)

    @pl.when(pl.program_id(0) == 0)
    def _():
        pooled_ref[...] = jnp.zeros_like(pooled_ref)
        cnts_ref[...] = jnp.zeros_like(cnts_ref)

    dnum = (((0,), (0,)), ((), ()))
    pooled_ref[...] += lax.dot_general(
        maskT, g, dnum, precision=lax.Precision.HIGHEST,
        preferred_element_type=jnp.float32)
    cnts_ref[...] += lax.dot_general(
        maskT, jnp.ones((BLK, 1), jnp.float32), dnum,
        precision=lax.Precision.HIGHEST,
        preferred_element_type=jnp.float32)


def _pool(acc, hprev, exself, b, batch2d):
    n, hw = hprev.shape
    return pl.pallas_call(
        _pool_body,
        grid=(n // BLK,),
        in_specs=[
            pl.BlockSpec((2, BLK, hw + L), lambda i: (0, i, 0)),
            pl.BlockSpec((BLK, hw), lambda i: (i, 0)),
            pl.BlockSpec((BLK, 1), lambda i: (i, 0)),
            pl.BlockSpec((1, hw), lambda i: (0, 0)),
            pl.BlockSpec((BLK, 1), lambda i: (i, 0)),
        ],
        out_specs=[
            pl.BlockSpec((G, hw), lambda i: (0, 0)),
            pl.BlockSpec((G, 1), lambda i: (0, 0)),
        ],
        out_shape=[
            jax.ShapeDtypeStruct((G, hw), jnp.float32),
            jax.ShapeDtypeStruct((G, 1), jnp.float32),
        ],
    )(acc, hprev, exself, b, batch2d)


def _head_body(pooled_ref, cnts_ref, wc1_ref, bc1_ref, wc2_ref, bc2_ref,
               out_ref):
    mean = pooled_ref[...] / jnp.maximum(cnts_ref[...], 1.0)
    z = jnp.maximum(_dot(mean, wc1_ref[...]) + bc1_ref[...], 0.0)
    out_ref[...] = _dot(z, wc2_ref[...]) + bc2_ref[...]


def _head(pooled, cnts, Wc1, bc1, Wc2, bc2):
    return pl.pallas_call(
        _head_body,
        out_shape=jax.ShapeDtypeStruct((G, Wc2.shape[1]), jnp.float32),
    )(pooled, cnts, Wc1, bc1, Wc2, bc2)


def _sc_edge(h, src, dst, a_s, a_d, m16, z80):
    n, hw = h.shape
    wide = hw + L
    e = src.shape[0]
    epw = e // NW
    ch = 80
    nchunk = epw // ch
    nbuf = 5
    nouter = nchunk // nbuf

    def body(h_hbm, src_hbm, dst_hbm, as_hbm, ad_hbm, m_hbm, z80_hbm,
             acc_out,
             asv, adv, mvv, srcv, dstv, hrows, rows80, acc_sh,
             sidx, sgat, ssct):
        ci = lax.axis_index("c")
        sax = lax.axis_index("s")

        @pl.when(sax == 0)
        def _():
            pltpu.sync_copy(z80_hbm, acc_sh)

        pltpu.sync_copy(as_hbm, asv)
        pltpu.sync_copy(ad_hbm, adv)
        pltpu.sync_copy(m_hbm, mvv)
        zv = jnp.zeros((L,), jnp.float32)
        for j in range(nbuf):
            @pl.loop(0, ch)
            def _zpad(ee):
                rows80[j, ee, pl.ds(hw, L)] = zv
        plsc.subcore_barrier()
        mvec = mvv[...]
        base0 = (ci * NS + sax) * epw

        def issue_idx(c, j):
            off = base0 + c * ch
            pltpu.make_async_copy(
                src_hbm.at[pl.ds(off, ch)], srcv.at[j], sidx.at[j]).start()
            pltpu.make_async_copy(
                dst_hbm.at[pl.ds(off, ch)], dstv.at[j], sidx.at[j]).start()

        def wait_idx(j):
            pltpu.make_async_copy(
                src_hbm.at[pl.ds(0, ch)], srcv.at[j], sidx.at[j]).wait()
            pltpu.make_async_copy(
                dst_hbm.at[pl.ds(0, ch)], dstv.at[j], sidx.at[j]).wait()

        def issue_gather(j):
            pltpu.make_async_copy(
                h_hbm.at[srcv.at[j]], hrows.at[j], sgat.at[j]).start()

        def wait_gather(j):
            pltpu.make_async_copy(
                h_hbm.at[srcv.at[j]], hrows.at[j], sgat.at[j]).wait()

        def issue_scatter(j):
            pltpu.make_async_copy(
                rows80.at[j], acc_sh.at[dstv.at[j]], ssct.at[j]).start(add=True)

        def wait_scatter(j):
            pltpu.make_async_copy(
                rows80.at[j], acc_sh.at[dstv.at[j]], ssct.at[j]).wait()

        def compute(j):
            @pl.loop(0, ch // L)
            def _cmp(i):
                s16 = srcv[j, pl.ds(i * L, L)]
                d16 = dstv[j, pl.ds(i * L, L)]
                zz = plsc.load_gather(asv, [s16]) + plsc.load_gather(adv, [d16])
                ex16 = jnp.exp(jnp.maximum(zz, 0.2 * zz) - mvec)
                eidx = lax.iota(jnp.int32, L) + i * L
                c64 = jnp.full((L,), hw, jnp.int32)
                plsc.store_scatter(rows80.at[j], [eidx, c64], ex16)
                for k in range(L):
                    exb = jnp.full((L,), 0.0, jnp.float32) + ex16[k]
                    for f in range(hw // L):
                        rows80[j, i * L + k, pl.ds(f * L, L)] = (
                            hrows[j, i * L + k, pl.ds(f * L, L)] * exb)

        issue_idx(0, 0)
        issue_idx(1, 1)
        wait_idx(0)
        issue_gather(0)

        @pl.loop(0, nouter)
        def _outer(t):
            for j5 in range(nbuf):
                jb = j5
                jb1 = (j5 + 1) % nbuf
                jb2 = (j5 + 2) % nbuf
                c = t * nbuf + j5

                @pl.when(c >= 3)
                def _():
                    wait_scatter(jb2)

                @pl.when(c + 2 < nchunk)
                def _():
                    issue_idx(c + 2, jb2)

                @pl.when(c + 1 < nchunk)
                def _():
                    wait_idx(jb1)
                    issue_gather(jb1)

                wait_gather(jb)
                compute(jb)
                issue_scatter(jb)

        for j in ((nchunk - 3) % nbuf, (nchunk - 2) % nbuf,
                  (nchunk - 1) % nbuf):
            wait_scatter(j)
        plsc.subcore_barrier()

        @pl.when(sax == 0)
        def _():
            pltpu.sync_copy(acc_sh, acc_out.at[ci])

    f = pl.kernel(
        body,
        out_type=jax.ShapeDtypeStruct((NC, n, wide), jnp.float32),
        mesh=plsc.VectorSubcoreMesh(core_axis_name="c", subcore_axis_name="s"),
        compiler_params=pltpu.CompilerParams(
            needs_layout_passes=False, use_tc_tiling_on_sc=False),
        scratch_types=[
            pltpu.VMEM((n,), jnp.float32),
            pltpu.VMEM((n,), jnp.float32),
            pltpu.VMEM((L,), jnp.float32),
            pltpu.VMEM((nbuf, ch), jnp.int32),
            pltpu.VMEM((nbuf, ch), jnp.int32),
            pltpu.VMEM((nbuf, ch, hw), jnp.float32),
            pltpu.VMEM((nbuf, ch, wide), jnp.float32),
            pltpu.VMEM_SHARED((n, wide), jnp.float32),
            pltpu.SemaphoreType.DMA((nbuf,)),
            pltpu.SemaphoreType.DMA((nbuf,)),
            pltpu.SemaphoreType.DMA((nbuf,)),
        ],
    )
    return f(h, src, dst, a_s, a_d, m16, z80)


def kernel(x, edge_index, batch, W1, a_src1, a_dst1, b1,
           W2, a_src2, a_dst2, b2, Wc1, bc1, Wc2, bc2):
    n, d = x.shape
    hw = W1.shape[1]
    src = edge_index[0]
    dst = edge_index[1]
    z80 = jnp.zeros((n, hw + L), jnp.float32)

    h1, as1, ad1 = _proj1(x, W1, a_src1.reshape(hw, 1), a_dst1.reshape(hw, 1))
    m1, exself1 = _selfw(as1, ad1)
    m16 = jnp.broadcast_to(m1.reshape(1), (L,))
    acc1 = _sc_edge(h1, src, dst, as1.reshape(n), ad1.reshape(n), m16, z80)
    h2, as2, ad2 = _proj2(acc1, h1, exself1, b1.reshape(1, hw),
                          W2, a_src2.reshape(hw, 1), a_dst2.reshape(hw, 1))
    m2, exself2 = _selfw(as2, ad2)
    m16b = jnp.broadcast_to(m2.reshape(1), (L,))
    acc2 = _sc_edge(h2, src, dst, as2.reshape(n), ad2.reshape(n), m16b, z80)
    pooled, cnts = _pool(acc2, h2, exself2, b2.reshape(1, hw),
                         batch.reshape(n, 1))
    return _head(pooled, cnts, Wc1, bc1.reshape(1, -1), Wc2, bc2.reshape(1, 1))
